# baseline (speedup 1.0000x reference)
import jax
import jax.numpy as jnp
from jax import lax
from jax.experimental import pallas as pl
from jax.experimental.pallas import tpu as pltpu
from jax.experimental.pallas import tpu_sc as plsc

B, S, D = 4, 4096, 2048
NC, NS = 2, 16
NW = NC * NS
DW = 256
ND = D // DW
NV = DW // 16
S_CHUNK = 128
N_CHUNK = S // S_CHUNK
NB = 3


def _scan_chunk_inplace(buf, carry):
    def row(s, carry):
        new = []
        for j in range(NV):
            x = buf[s, pl.ds(j * 16, 16)]
            acc = carry[j] + x
            buf[s, pl.ds(j * 16, 16)] = acc
            new.append(acc)
        return tuple(new)

    return lax.fori_loop(0, S_CHUNK, row, carry)


def _cumsum_body(val_hbm, out_hbm, bufs, sls, sss):
    wid = lax.axis_index("s") * NC + lax.axis_index("c")
    b = wid // ND
    d0 = pl.multiple_of((wid % ND) * DW, DW)

    def hbm(c):
        return val_hbm.at[b, pl.ds(c * S_CHUNK, S_CHUNK), pl.ds(d0, DW)], \
               out_hbm.at[b, pl.ds(c * S_CHUNK, S_CHUNK), pl.ds(d0, DW)]

    def start_load(c):
        pltpu.async_copy(hbm(c)[0], bufs[c % NB], sls[c % NB])

    def wait_load(c):
        pltpu.make_async_copy(hbm(c)[0], bufs[c % NB], sls[c % NB]).wait()

    def start_store(c):
        pltpu.async_copy(bufs[c % NB], hbm(c)[1], sss[c % NB])

    def wait_store(c):
        pltpu.make_async_copy(bufs[c % NB], hbm(c)[1], sss[c % NB]).wait()

    for c in range(NB - 1):
        start_load(c)
    carry = tuple(jnp.zeros((16,), jnp.float32) for _ in range(NV))
    for c in range(N_CHUNK):
        wait_load(c)
        carry = _scan_chunk_inplace(bufs[c % NB], carry)
        start_store(c)
        if c + NB - 1 < N_CHUNK:
            if c > 0:
                wait_store(c - 1)
            start_load(c + NB - 1)
    for c in range(N_CHUNK - NB, N_CHUNK):
        wait_store(c)


@jax.jit
def kernel(val):
    mesh = plsc.VectorSubcoreMesh(core_axis_name="c", subcore_axis_name="s")

    def body(val_hbm, out_hbm, b0, b1, b2, sl0, sl1, sl2, ss0, ss1, ss2):
        _cumsum_body(val_hbm, out_hbm, (b0, b1, b2), (sl0, sl1, sl2),
                     (ss0, ss1, ss2))

    f = pl.kernel(
        body,
        out_type=jax.ShapeDtypeStruct((B, S, D), jnp.float32),
        mesh=mesh,
        scratch_types=(
            [pltpu.VMEM((S_CHUNK, DW), jnp.float32)] * NB
            + [pltpu.SemaphoreType.DMA] * (2 * NB)
        ),
    )
    return f(val)
